# baseline (speedup 1.0000x reference)
import functools

import jax
import jax.numpy as jnp
from jax import lax
from jax.experimental import pallas as pl
from jax.experimental.pallas import tpu as pltpu
from jax.experimental.pallas import tpu_sc as plsc

VOCAB = 1000
NB, NT = 1024, 50
NTOK = NB * NT
NC, NS = 2, 16
NW = NC * NS
NPW = NTOK // NW


def _prep_body(table_ref, idx_ref, lse_ref, tb_ref, idxT_ref):
    x = table_ref[...]
    m = jnp.max(x, axis=1, keepdims=True)
    s = jnp.sum(jnp.exp(x - m), axis=1, keepdims=True)
    lse_ref[...] = m + jnp.log(s)
    tb_ref[...] = x.astype(jnp.bfloat16)
    idxT_ref[...] = jnp.transpose(idx_ref[...], (1, 0))[:, None, :]


def _prep(table, idx32):
    return pl.pallas_call(
        _prep_body,
        out_shape=[jax.ShapeDtypeStruct((VOCAB, 1), jnp.float32),
                   jax.ShapeDtypeStruct((VOCAB, VOCAB), jnp.bfloat16),
                   jax.ShapeDtypeStruct((NT, 1, NB), jnp.int32)],
    )(table, idx32)


_MESH = plsc.VectorSubcoreMesh(core_axis_name="c", subcore_axis_name="s")


@functools.partial(
    pl.kernel,
    mesh=_MESH,
    out_type=jax.ShapeDtypeStruct((NW, 16), jnp.float32),
    scratch_types=[
        pltpu.VMEM((NPW,), jnp.int32),
        pltpu.VMEM((NPW,), jnp.int32),
        pltpu.VMEM((NPW,), jnp.float32),
        pltpu.VMEM((1024,), jnp.float32),
        pltpu.VMEM((16,), jnp.float32),
        pltpu.SemaphoreType.DMA,
    ],
    compiler_params=pltpu.CompilerParams(use_tc_tiling_on_sc=False,
                                         needs_layout_passes=False),
)
def _sc_loss(tablef_hbm, packed_hbm, lse_hbm, part_hbm,
             packed_v, fidx_v, vals_v, lse_v, acc_v, sem):
    wid = lax.axis_index("s") * NC + lax.axis_index("c")
    base = wid * NPW
    pltpu.sync_copy(lse_hbm, lse_v)
    pltpu.sync_copy(packed_hbm.at[pl.ds(base, NPW)], packed_v)

    for i in range(NPW // 16):
        s = pl.ds(i * 16, 16)
        p = packed_v[s]
        fidx_v[s] = (lax.shift_right_logical(p, 10) * VOCAB
                     + lax.bitwise_and(p, 1023))

    descs = []
    for k in range(13):
        off = k * 128
        n = 128 if off + 128 <= NPW else NPW - off
        descs.append(pltpu.async_copy(
            tablef_hbm.at[fidx_v.at[pl.ds(off, n)]],
            vals_v.at[pl.ds(off, n)], sem))
    for d in descs:
        d.wait()

    acc = jnp.zeros((16,), jnp.float32)
    for i in range(NPW // 16):
        s = pl.ds(i * 16, 16)
        lses = plsc.load_gather(
            lse_v, [lax.shift_right_logical(packed_v[s], 10)])
        acc = acc + (lses - vals_v[s])
    acc_v[...] = acc
    pltpu.sync_copy(acc_v, part_hbm.at[wid])


def _gather_body(idx_ref, tbT_ref, out_ref):
    idxr = idx_ref[0]
    iota = lax.broadcasted_iota(jnp.int32, (VOCAB, NB), 0)
    oh = jnp.where(iota == idxr, 1.0, 0.0).astype(jnp.bfloat16)
    out_ref[0] = jax.lax.dot_general(
        tbT_ref[...], oh, (((0,), (0,)), ((), ())),
        preferred_element_type=jnp.float32)


def _mxu_gather(idxT3, tbT):
    return pl.pallas_call(
        _gather_body,
        grid=(NT,),
        in_specs=[
            pl.BlockSpec((1, 1, NB), lambda t: (t, 0, 0)),
            pl.BlockSpec((VOCAB, VOCAB), lambda t: (0, 0)),
        ],
        out_specs=pl.BlockSpec((1, VOCAB, NB), lambda t: (t, 0, 0)),
        out_shape=jax.ShapeDtypeStruct((NT, VOCAB, NB), jnp.float32),
    )(idxT3, tbT)


def _loss_body(part_ref, out_ref):
    out_ref[...] = (jnp.sum(part_ref[...]) / NTOK).reshape(1, 1)


def _loss_reduce(partials):
    return pl.pallas_call(
        _loss_body,
        out_shape=jax.ShapeDtypeStruct((1, 1), jnp.float32),
    )(partials)


def kernel(idx, targets, table):
    idx32 = idx.astype(jnp.int32)
    packed = (idx32.reshape(-1) * 1024
              + targets.reshape(-1).astype(jnp.int32))
    table = table.astype(jnp.float32)
    lse, tb, idxT3 = _prep(table, idx32)
    lse_pad = jnp.pad(lse.reshape(VOCAB), (0, 1024 - VOCAB))
    tablef = table.reshape(VOCAB * VOCAB)

    part = _sc_loss(tablef, packed, lse_pad)
    out = _mxu_gather(idxT3, tb)
    logits = jnp.transpose(out, (2, 0, 1))
    loss = _loss_reduce(part)[0, 0]
    return (logits, loss)


# speedup vs baseline: 1.0473x; 1.0473x over previous
import functools

import jax
import jax.numpy as jnp
from jax import lax
from jax.experimental import pallas as pl
from jax.experimental.pallas import tpu as pltpu
from jax.experimental.pallas import tpu_sc as plsc

VOCAB = 1000
NB, NT = 1024, 50
NTOK = NB * NT
NC, NS = 2, 16
NW = NC * NS
NPW = NTOK // NW


def _prep_body(table_ref, lse_ref):
    x = table_ref[...]
    m = jnp.max(x, axis=1, keepdims=True)
    s = jnp.sum(jnp.exp(x - m), axis=1, keepdims=True)
    lse_ref[...] = m + jnp.log(s)


def _prep(table):
    return pl.pallas_call(
        _prep_body,
        out_shape=jax.ShapeDtypeStruct((VOCAB, 1), jnp.float32),
    )(table)


_MESH = plsc.VectorSubcoreMesh(core_axis_name="c", subcore_axis_name="s")


@functools.partial(
    pl.kernel,
    mesh=_MESH,
    out_type=jax.ShapeDtypeStruct((NW, 16), jnp.float32),
    scratch_types=[
        pltpu.VMEM((NPW,), jnp.int32),
        pltpu.VMEM((NPW,), jnp.int32),
        pltpu.VMEM((NPW,), jnp.float32),
        pltpu.VMEM((1024,), jnp.float32),
        pltpu.VMEM((16,), jnp.float32),
        pltpu.SemaphoreType.DMA,
    ],
    compiler_params=pltpu.CompilerParams(use_tc_tiling_on_sc=False,
                                         needs_layout_passes=False),
)
def _sc_loss(tablef_hbm, packed_hbm, lse_hbm, part_hbm,
             packed_v, fidx_v, vals_v, lse_v, acc_v, sem):
    wid = lax.axis_index("s") * NC + lax.axis_index("c")
    base = wid * NPW
    pltpu.sync_copy(lse_hbm, lse_v)
    pltpu.sync_copy(packed_hbm.at[pl.ds(base, NPW)], packed_v)

    for i in range(NPW // 16):
        s = pl.ds(i * 16, 16)
        p = packed_v[s]
        fidx_v[s] = (lax.shift_right_logical(p, 10) * VOCAB
                     + lax.bitwise_and(p, 1023))

    descs = []
    for k in range(13):
        off = k * 128
        n = 128 if off + 128 <= NPW else NPW - off
        descs.append(pltpu.async_copy(
            tablef_hbm.at[fidx_v.at[pl.ds(off, n)]],
            vals_v.at[pl.ds(off, n)], sem))
    for d in descs:
        d.wait()

    acc = jnp.zeros((16,), jnp.float32)
    for i in range(NPW // 16):
        s = pl.ds(i * 16, 16)
        lses = plsc.load_gather(
            lse_v, [lax.shift_right_logical(packed_v[s], 10)])
        acc = acc + (lses - vals_v[s])
    acc_v[...] = acc
    pltpu.sync_copy(acc_v, part_hbm.at[wid])


TB = 2


def _gather_body(idx_ref, tbT_ref, out_ref):
    iota = lax.broadcasted_iota(jnp.int32, (VOCAB, NB), 0)
    for k in range(TB):
        idxr = idx_ref[k]
        oh = jnp.where(iota == idxr, 1.0, 0.0).astype(jnp.bfloat16)
        out_ref[k] = jax.lax.dot_general(
            tbT_ref[...], oh, (((1,), (0,)), ((), ())),
            preferred_element_type=jnp.float32)


def _mxu_gather(idxT3, tbT):
    return pl.pallas_call(
        _gather_body,
        grid=(NT // TB,),
        in_specs=[
            pl.BlockSpec((TB, 1, NB), lambda t: (t, 0, 0)),
            pl.BlockSpec((VOCAB, VOCAB), lambda t: (0, 0)),
        ],
        out_specs=pl.BlockSpec((TB, VOCAB, NB), lambda t: (t, 0, 0)),
        out_shape=jax.ShapeDtypeStruct((NT, VOCAB, NB), jnp.float32),
    )(idxT3, tbT)


def _loss_body(part_ref, out_ref):
    out_ref[...] = (jnp.sum(part_ref[...]) / NTOK).reshape(1, 1)


def _loss_reduce(partials):
    return pl.pallas_call(
        _loss_body,
        out_shape=jax.ShapeDtypeStruct((1, 1), jnp.float32),
    )(partials)


def kernel(idx, targets, table):
    idx32 = idx.astype(jnp.int32)
    packed = (idx32.reshape(-1) * 1024
              + targets.reshape(-1).astype(jnp.int32))
    table = table.astype(jnp.float32)
    idxT3 = idx32.T.reshape(NT, 1, NB)
    tbT = table.T.astype(jnp.bfloat16)

    lse = _prep(table)
    lse_pad = jnp.pad(lse.reshape(VOCAB), (0, 1024 - VOCAB))
    tablef = table.reshape(VOCAB * VOCAB)

    part = _sc_loss(tablef, packed, lse_pad)
    out = _mxu_gather(idxT3, tbT)
    logits = jnp.transpose(out, (2, 0, 1))
    loss = _loss_reduce(part)[0, 0]
    return (logits, loss)
